# Initial kernel scaffold; baseline (speedup 1.0000x reference)
import jax
import jax.numpy as jnp
from jax.experimental import pallas as pl


def kernel(mem, idx, val):
    raise NotImplementedError("write your pallas kernel here")


# speedup vs baseline: 50.2260x; 50.2260x over previous
import functools

import jax
import jax.numpy as jnp
from jax import lax
from jax.experimental import pallas as pl
from jax.experimental.pallas import tpu as pltpu
from jax.experimental.pallas import tpu_sc as plsc

M = 1048576
D = 64
B = 65536

GR = 128
NC = 2
NS = 16
NW = NC * NS
NGRP = B // GR
P1_GRPS = NGRP // NS
P1_CHUNK = B // NS
FLEN = P1_CHUNK + GR
P2_GRPS = NGRP // NW
P2_CHUNK = B // NW
NBUF = 2
GRH = 64
NH = P2_CHUNK // GRH
WPAD = NS * GR


def _body(idx_hbm, jv_hbm, val2_hbm, out_hbm,
          idxs_v, jvs_v, wjs_v, fk_v, fj_v, stg_v, oidx_v, wj_v, pidx_v,
          p0_v, p1_v, ob0_v, ob1_v, w_sh,
          sem_a, sem_b, gsems, wsems):
  c = lax.axis_index("c")
  s = lax.axis_index("s")

  pltpu.sync_copy(idx_hbm.at[pl.ds(s * P1_GRPS, P1_GRPS)], idxs_v)
  pltpu.sync_copy(jv_hbm.at[pl.ds(s * P1_GRPS, P1_GRPS)], jvs_v)

  lanes = lax.iota(jnp.int32, 16)

  def prefill(i, carry):
    dvec = (M + s * GR + (i % 8) * 16) + lanes
    fk_v[pl.ds(i * 16, 16)] = dvec
    return carry
  lax.fori_loop(0, FLEN // 16, prefill, 0)

  cps = []
  for r in range(P1_GRPS):
    cps.append(pltpu.async_copy(jvs_v.at[r], w_sh.at[idxs_v.at[r]], sem_a))
  for cp in cps:
    cp.wait()
  plsc.subcore_barrier()

  cps = []
  for r in range(P1_GRPS):
    cps.append(pltpu.async_copy(w_sh.at[idxs_v.at[r]], wjs_v.at[r], sem_b))
  for cp in cps:
    cp.wait()

  def compact(i, cnt):
    r = i // 8
    col = (i % 8) * 16
    kv = idxs_v[r, pl.ds(col, 16)]
    jv = jvs_v[r, pl.ds(col, 16)]
    wv = wjs_v[r, pl.ds(col, 16)]
    m = wv < jv
    mi = m.astype(jnp.int32)
    excl = plsc.cumsum(mi) - mi
    dest = excl + cnt
    plsc.store_scatter(fk_v, [dest], kv, mask=m)
    plsc.store_scatter(fj_v, [dest], jv, mask=m)
    return cnt + jnp.sum(mi)
  cnt = lax.fori_loop(0, P1_CHUNK // 16, compact, 0)
  ngrp = (cnt + GR - 1) // GR

  for t in range(NS):
    @pl.when(s == t)
    def _():
      def fix(g, carry):
        for k in range(GR // 16):
          stg_v[0, pl.ds(k * 16, 16)] = fk_v[pl.ds(g * GR + k * 16, 16)]
        pltpu.sync_copy(fj_v.at[pl.ds(g * GR, GR)], w_sh.at[stg_v.at[0]])
        return carry
      lax.fori_loop(0, ngrp, fix, 0)
    plsc.subcore_barrier()

  w = s * NC + c
  pltpu.sync_copy(idx_hbm.at[pl.ds(w * P2_GRPS, P2_GRPS)], oidx_v)
  cps = []
  for r in range(P2_GRPS):
    cps.append(pltpu.async_copy(w_sh.at[oidx_v.at[r]], wj_v.at[r], sem_a))
  for cp in cps:
    cp.wait()

  def mkpidx(i, carry):
    r = i // 4
    col = (i % 4) * 16
    rr = i // 8
    cc = (i % 8) * 16
    pidx_v[r, pl.ds(col, 16)] = jax.lax.shift_right_logical(
        wj_v[rr, pl.ds(cc, 16)], 1)
    return carry
  lax.fori_loop(0, P2_CHUNK // 16, mkpidx, 0)

  pbufs = [p0_v, p1_v]
  obufs = [ob0_v, ob1_v]

  def select_group(h, pbuf, obuf):
    wr = h // 2
    wc = (h % 2) * GRH
    def blk(b, carry):
      offv = (wj_v[wr, pl.ds(wc + b * 16, 16)] & 1) * D
      for l in range(16):
        i = b * 16 + l
        off = offv[l]
        for k in range(D // 16):
          obuf[i, pl.ds(k * 16, 16)] = pbuf[i, pl.ds(off + k * 16, 16)]
      return carry
    lax.fori_loop(0, GRH // 16, blk, 0)

  def emit_write(h):
    return pltpu.async_copy(
        obufs[h % 2],
        out_hbm.at[pl.ds(w * P2_CHUNK + h * GRH, GRH)],
        wsems.at[h % 2])

  gcps = [None] * NH
  wcps = [None] * NH
  for h in range(NH):
    gcps[h] = pltpu.async_copy(
        val2_hbm.at[pidx_v.at[h]], pbufs[h % NBUF], gsems.at[h % NBUF])
    if h >= 1:
      hp = h - 1
      gcps[hp].wait()
      if hp >= 2:
        wcps[hp - 2].wait()
      select_group(hp, pbufs[hp % NBUF], obufs[hp % 2])
      wcps[hp] = emit_write(hp)
  hl = NH - 1
  gcps[hl].wait()
  wcps[hl - 2].wait()
  select_group(hl, pbufs[hl % NBUF], obufs[hl % 2])
  wcps[hl] = emit_write(hl)
  wcps[hl - 1].wait()
  wcps[hl].wait()


@jax.jit
def _run(idx2d, jv2d, val2):
  mesh = plsc.VectorSubcoreMesh(core_axis_name="c", subcore_axis_name="s")
  kfn = functools.partial(
      pl.kernel,
      mesh=mesh,
      compiler_params=pltpu.CompilerParams(needs_layout_passes=False),
      out_type=jax.ShapeDtypeStruct((B, D), jnp.float32),
      scratch_types=[
          pltpu.VMEM((P1_GRPS, GR), jnp.int32),
          pltpu.VMEM((P1_GRPS, GR), jnp.int32),
          pltpu.VMEM((P1_GRPS, GR), jnp.int32),
          pltpu.VMEM((FLEN,), jnp.int32),
          pltpu.VMEM((FLEN,), jnp.int32),
          pltpu.VMEM((1, GR), jnp.int32),
          pltpu.VMEM((P2_GRPS, GR), jnp.int32),
          pltpu.VMEM((P2_GRPS, GR), jnp.int32),
          pltpu.VMEM((NH, GRH), jnp.int32),
          pltpu.VMEM((GRH, 2 * D), jnp.float32),
          pltpu.VMEM((GRH, 2 * D), jnp.float32),
          pltpu.VMEM((GRH, D), jnp.float32),
          pltpu.VMEM((GRH, D), jnp.float32),
          pltpu.VMEM_SHARED((M + WPAD,), jnp.int32),
          pltpu.SemaphoreType.DMA,
          pltpu.SemaphoreType.DMA,
          pltpu.SemaphoreType.DMA((NBUF,)),
          pltpu.SemaphoreType.DMA((2,)),
      ],
  )(_body)
  return kfn(idx2d, jv2d, val2)


def kernel(mem, idx, val):
  del mem
  idx2d = idx.reshape(NGRP, GR)
  jv2d = jnp.arange(B, dtype=jnp.int32).reshape(NGRP, GR)
  val2 = val.reshape(B // 2, 2 * D)
  return _run(idx2d, jv2d, val2)


# speedup vs baseline: 55.3512x; 1.1020x over previous
import functools

import jax
import jax.numpy as jnp
from jax import lax
from jax.experimental import pallas as pl
from jax.experimental.pallas import tpu as pltpu
from jax.experimental.pallas import tpu_sc as plsc

M = 1048576
D = 64
B = 65536

GR = 128
NC = 2
NS = 16
NW = NC * NS
NGRP = B // GR
P1_GRPS = NGRP // NS
P1_CHUNK = B // NS
FLEN = P1_CHUNK + GR
P2_GRPS = NGRP // NW
P2_CHUNK = B // NW
NBUF = 4
WPAD = NS * GR


def _body(idx_hbm, jv_hbm, val_hbm, out_hbm,
          idxs_v, jvs_v, wjs_v, fk_v, fj_v, stg_v, oidx_v, wj_v,
          r0_v, r1_v, r2_v, r3_v, w_sh,
          sem_a, sem_b, gsems, wsems):
  c = lax.axis_index("c")
  s = lax.axis_index("s")

  pltpu.sync_copy(idx_hbm.at[pl.ds(s * P1_GRPS, P1_GRPS)], idxs_v)
  pltpu.sync_copy(jv_hbm.at[pl.ds(s * P1_GRPS, P1_GRPS)], jvs_v)

  lanes = lax.iota(jnp.int32, 16)

  def prefill(i, carry):
    dvec = (M + s * GR + (i % 8) * 16) + lanes
    fk_v[pl.ds(i * 16, 16)] = dvec
    return carry
  lax.fori_loop(0, FLEN // 16, prefill, 0)

  cps = []
  for r in range(P1_GRPS):
    cps.append(pltpu.async_copy(jvs_v.at[r], w_sh.at[idxs_v.at[r]], sem_a))
  for cp in cps:
    cp.wait()
  plsc.subcore_barrier()

  cps = []
  for r in range(P1_GRPS):
    cps.append(pltpu.async_copy(w_sh.at[idxs_v.at[r]], wjs_v.at[r], sem_b))
  for cp in cps:
    cp.wait()

  def compact(i, cnt):
    r = i // 8
    col = (i % 8) * 16
    kv = idxs_v[r, pl.ds(col, 16)]
    jv = jvs_v[r, pl.ds(col, 16)]
    wv = wjs_v[r, pl.ds(col, 16)]
    m = wv < jv
    mi = m.astype(jnp.int32)
    excl = plsc.cumsum(mi) - mi
    dest = excl + cnt
    plsc.store_scatter(fk_v, [dest], kv, mask=m)
    plsc.store_scatter(fj_v, [dest], jv, mask=m)
    return cnt + jnp.sum(mi)
  cnt = lax.fori_loop(0, P1_CHUNK // 16, compact, 0)
  ngrp = (cnt + GR - 1) // GR

  for t in range(NS):
    @pl.when(s == t)
    def _():
      def fix(g, carry):
        for k in range(GR // 16):
          stg_v[0, pl.ds(k * 16, 16)] = fk_v[pl.ds(g * GR + k * 16, 16)]
        pltpu.sync_copy(fj_v.at[pl.ds(g * GR, GR)], w_sh.at[stg_v.at[0]])
        return carry
      lax.fori_loop(0, ngrp, fix, 0)
    plsc.subcore_barrier()

  w = s * NC + c
  pltpu.sync_copy(idx_hbm.at[pl.ds(w * P2_GRPS, P2_GRPS)], oidx_v)
  cps = []
  for r in range(P2_GRPS):
    cps.append(pltpu.async_copy(w_sh.at[oidx_v.at[r]], wj_v.at[r], sem_a))
  for cp in cps:
    cp.wait()

  bufs = [r0_v, r1_v, r2_v, r3_v]
  gcps = [None] * P2_GRPS
  wcps = [None] * P2_GRPS
  for g in range(P2_GRPS):
    if g >= NBUF:
      wcps[g - NBUF].wait()
    gcps[g] = pltpu.async_copy(
        val_hbm.at[wj_v.at[g]], bufs[g % NBUF], gsems.at[g % NBUF])
    if g >= 1:
      gcps[g - 1].wait()
      wcps[g - 1] = pltpu.async_copy(
          bufs[(g - 1) % NBUF],
          out_hbm.at[w * P2_GRPS + (g - 1)],
          wsems.at[(g - 1) % NBUF])
  gl = P2_GRPS - 1
  gcps[gl].wait()
  wcps[gl] = pltpu.async_copy(
      bufs[gl % NBUF], out_hbm.at[w * P2_GRPS + gl], wsems.at[gl % NBUF])
  for g in range(P2_GRPS - NBUF + 1, P2_GRPS + 1):
    wcps[g - 1].wait()


@jax.jit
def _run(idx2d, jv2d, val):
  mesh = plsc.VectorSubcoreMesh(core_axis_name="c", subcore_axis_name="s")
  kfn = functools.partial(
      pl.kernel,
      mesh=mesh,
      compiler_params=pltpu.CompilerParams(
          use_tc_tiling_on_sc=False, needs_layout_passes=False),
      out_type=jax.ShapeDtypeStruct((NGRP, GR, D), jnp.float32),
      scratch_types=[
          pltpu.VMEM((P1_GRPS, GR), jnp.int32),
          pltpu.VMEM((P1_GRPS, GR), jnp.int32),
          pltpu.VMEM((P1_GRPS, GR), jnp.int32),
          pltpu.VMEM((FLEN,), jnp.int32),
          pltpu.VMEM((FLEN,), jnp.int32),
          pltpu.VMEM((1, GR), jnp.int32),
          pltpu.VMEM((P2_GRPS, GR), jnp.int32),
          pltpu.VMEM((P2_GRPS, GR), jnp.int32),
          pltpu.VMEM((GR, D), jnp.float32),
          pltpu.VMEM((GR, D), jnp.float32),
          pltpu.VMEM((GR, D), jnp.float32),
          pltpu.VMEM((GR, D), jnp.float32),
          pltpu.VMEM_SHARED((M + WPAD,), jnp.int32),
          pltpu.SemaphoreType.DMA,
          pltpu.SemaphoreType.DMA,
          pltpu.SemaphoreType.DMA((NBUF,)),
          pltpu.SemaphoreType.DMA((NBUF,)),
      ],
  )(_body)
  return kfn(idx2d, jv2d, val)


def kernel(mem, idx, val):
  del mem
  idx2d = idx.reshape(NGRP, GR)
  jv2d = jnp.arange(B, dtype=jnp.int32).reshape(NGRP, GR)
  return _run(idx2d, jv2d, val).reshape(B, D)


# speedup vs baseline: 59.3112x; 1.0715x over previous
import functools

import jax
import jax.numpy as jnp
from jax import lax
from jax.experimental import pallas as pl
from jax.experimental.pallas import tpu as pltpu
from jax.experimental.pallas import tpu_sc as plsc

M = 1048576
D = 64
B = 65536

GR = 128
NC = 2
NS = 16
NW = NC * NS
NGRP = B // GR
P1_GRPS = NGRP // NS
P1_CHUNK = B // NS
FLEN = P1_CHUNK + GR
P2_GRPS = NGRP // NW
P2_CHUNK = B // NW
NBUF = 4
WPAD = NS * GR


def _winners_body(idx_hbm, jv_hbm, wj_hbm,
                  idxs_v, jvs_v, wjs_v, fk_v, fj_v, stg_v, oidx_v, wj_v,
                  w_sh, sem_a, sem_b):
  c = lax.axis_index("c")
  s = lax.axis_index("s")

  pltpu.sync_copy(idx_hbm.at[pl.ds(s * P1_GRPS, P1_GRPS)], idxs_v)
  pltpu.sync_copy(jv_hbm.at[pl.ds(s * P1_GRPS, P1_GRPS)], jvs_v)

  lanes = lax.iota(jnp.int32, 16)

  def prefill(i, carry):
    dvec = (M + s * GR + (i % 8) * 16) + lanes
    fk_v[pl.ds(i * 16, 16)] = dvec
    return carry
  lax.fori_loop(0, FLEN // 16, prefill, 0)

  cps = []
  for r in range(P1_GRPS):
    cps.append(pltpu.async_copy(jvs_v.at[r], w_sh.at[idxs_v.at[r]], sem_a))
  for cp in cps:
    cp.wait()
  plsc.subcore_barrier()

  cps = []
  for r in range(P1_GRPS):
    cps.append(pltpu.async_copy(w_sh.at[idxs_v.at[r]], wjs_v.at[r], sem_b))
  for cp in cps:
    cp.wait()

  def compact(i, cnt):
    r = i // 8
    col = (i % 8) * 16
    kv = idxs_v[r, pl.ds(col, 16)]
    jv = jvs_v[r, pl.ds(col, 16)]
    wv = wjs_v[r, pl.ds(col, 16)]
    m = wv < jv
    mi = m.astype(jnp.int32)
    excl = plsc.cumsum(mi) - mi
    dest = excl + cnt
    plsc.store_scatter(fk_v, [dest], kv, mask=m)
    plsc.store_scatter(fj_v, [dest], jv, mask=m)
    return cnt + jnp.sum(mi)
  cnt = lax.fori_loop(0, P1_CHUNK // 16, compact, 0)
  ngrp = (cnt + GR - 1) // GR

  for t in range(NS):
    @pl.when(s == t)
    def _():
      def fix(g, carry):
        for k in range(GR // 16):
          stg_v[0, pl.ds(k * 16, 16)] = fk_v[pl.ds(g * GR + k * 16, 16)]
        pltpu.sync_copy(fj_v.at[pl.ds(g * GR, GR)], w_sh.at[stg_v.at[0]])
        return carry
      lax.fori_loop(0, ngrp, fix, 0)
    plsc.subcore_barrier()

  w = s * NC + c
  pltpu.sync_copy(idx_hbm.at[pl.ds(w * P2_GRPS, P2_GRPS)], oidx_v)
  cps = []
  for r in range(P2_GRPS):
    cps.append(pltpu.async_copy(w_sh.at[oidx_v.at[r]], wj_v.at[r], sem_a))
  for cp in cps:
    cp.wait()
  pltpu.sync_copy(wj_v, wj_hbm.at[pl.ds(w * P2_GRPS, P2_GRPS)])


def _permute_body(wj_hbm, val2_hbm, out_hbm,
                  wj_v, pidx_v, p0_v, p1_v, p2_v, p3_v, ob0_v, ob1_v,
                  gsems, wsems):
  c = lax.axis_index("c")
  s = lax.axis_index("s")
  w = s * NC + c

  pltpu.sync_copy(wj_hbm.at[pl.ds(w * P2_GRPS, P2_GRPS)], wj_v)

  def mkpidx(i, carry):
    r = i // 8
    col = (i % 8) * 16
    pidx_v[r, pl.ds(col, 16)] = jax.lax.shift_right_logical(
        wj_v[r, pl.ds(col, 16)], 1)
    return carry
  lax.fori_loop(0, P2_CHUNK // 16, mkpidx, 0)

  pbufs = [p0_v, p1_v, p2_v, p3_v]
  obufs = [ob0_v, ob1_v]

  def select_group(g, pbuf, obuf):
    def blk(b, carry):
      offv = (wj_v[g, pl.ds(b * 16, 16)] & 1) * D
      for l in range(16):
        i = b * 16 + l
        off = offv[l]
        for k in range(D // 16):
          obuf[i, pl.ds(k * 16, 16)] = pbuf[i, pl.ds(off + k * 16, 16)]
      return carry
    lax.fori_loop(0, GR // 16, blk, 0)

  gcps = [None] * P2_GRPS
  wcps = [None] * P2_GRPS
  for g in range(P2_GRPS):
    gcps[g] = pltpu.async_copy(
        val2_hbm.at[pidx_v.at[g]], pbufs[g % NBUF], gsems.at[g % NBUF])
    if g >= 1:
      gp = g - 1
      gcps[gp].wait()
      if gp >= 2:
        wcps[gp - 2].wait()
      select_group(gp, pbufs[gp % NBUF], obufs[gp % 2])
      wcps[gp] = pltpu.async_copy(
          obufs[gp % 2],
          out_hbm.at[pl.ds(w * P2_CHUNK + gp * GR, GR)],
          wsems.at[gp % 2])
  gl = P2_GRPS - 1
  gcps[gl].wait()
  wcps[gl - 2].wait()
  select_group(gl, pbufs[gl % NBUF], obufs[gl % 2])
  wcps[gl] = pltpu.async_copy(
      obufs[gl % 2],
      out_hbm.at[pl.ds(w * P2_CHUNK + gl * GR, GR)],
      wsems.at[gl % 2])
  wcps[gl - 1].wait()
  wcps[gl].wait()


@jax.jit
def _run(idx2d, jv2d, val2):
  mesh = plsc.VectorSubcoreMesh(core_axis_name="c", subcore_axis_name="s")
  params = pltpu.CompilerParams(needs_layout_passes=False)
  winners = functools.partial(
      pl.kernel,
      mesh=mesh,
      compiler_params=params,
      out_type=jax.ShapeDtypeStruct((NGRP, GR), jnp.int32),
      scratch_types=[
          pltpu.VMEM((P1_GRPS, GR), jnp.int32),
          pltpu.VMEM((P1_GRPS, GR), jnp.int32),
          pltpu.VMEM((P1_GRPS, GR), jnp.int32),
          pltpu.VMEM((FLEN,), jnp.int32),
          pltpu.VMEM((FLEN,), jnp.int32),
          pltpu.VMEM((1, GR), jnp.int32),
          pltpu.VMEM((P2_GRPS, GR), jnp.int32),
          pltpu.VMEM((P2_GRPS, GR), jnp.int32),
          pltpu.VMEM_SHARED((M + WPAD,), jnp.int32),
          pltpu.SemaphoreType.DMA,
          pltpu.SemaphoreType.DMA,
      ],
  )(_winners_body)
  permute = functools.partial(
      pl.kernel,
      mesh=mesh,
      compiler_params=params,
      out_type=jax.ShapeDtypeStruct((B, D), jnp.float32),
      scratch_types=[
          pltpu.VMEM((P2_GRPS, GR), jnp.int32),
          pltpu.VMEM((P2_GRPS, GR), jnp.int32),
          pltpu.VMEM((GR, 2 * D), jnp.float32),
          pltpu.VMEM((GR, 2 * D), jnp.float32),
          pltpu.VMEM((GR, 2 * D), jnp.float32),
          pltpu.VMEM((GR, 2 * D), jnp.float32),
          pltpu.VMEM((GR, D), jnp.float32),
          pltpu.VMEM((GR, D), jnp.float32),
          pltpu.SemaphoreType.DMA((NBUF,)),
          pltpu.SemaphoreType.DMA((2,)),
      ],
  )(_permute_body)
  wj2d = winners(idx2d, jv2d)
  return permute(wj2d, val2)


def kernel(mem, idx, val):
  del mem
  idx2d = idx.reshape(NGRP, GR)
  jv2d = jnp.arange(B, dtype=jnp.int32).reshape(NGRP, GR)
  val2 = val.reshape(B // 2, 2 * D)
  return _run(idx2d, jv2d, val2)


# speedup vs baseline: 59.4525x; 1.0024x over previous
import functools

import jax
import jax.numpy as jnp
from jax import lax
from jax.experimental import pallas as pl
from jax.experimental.pallas import tpu as pltpu
from jax.experimental.pallas import tpu_sc as plsc

M = 1048576
D = 64
B = 65536

GR = 128
NC = 2
NS = 16
NW = NC * NS
NGRP = B // GR
P1_GRPS = NGRP // NS
P1_CHUNK = B // NS
FLEN = P1_CHUNK + GR
P2_GRPS = NGRP // NW
P2_CHUNK = B // NW
NBUF = 4
WPAD = NS * GR


def _winners_body(idx_hbm, jv_hbm, wj_hbm,
                  idxs_v, jvs_v, wjs_v, fk_v, fj_v, stg_v, oidx_v, wj_v,
                  w_sh, sem_a, sem_b):
  c = lax.axis_index("c")
  s = lax.axis_index("s")

  pltpu.sync_copy(idx_hbm.at[pl.ds(s * P1_GRPS, P1_GRPS)], idxs_v)
  pltpu.sync_copy(jv_hbm.at[pl.ds(s * P1_GRPS, P1_GRPS)], jvs_v)

  lanes = lax.iota(jnp.int32, 16)

  def prefill(i, carry):
    dvec = (M + s * GR + (i % 8) * 16) + lanes
    fk_v[pl.ds(i * 16, 16)] = dvec
    return carry
  lax.fori_loop(0, FLEN // 16, prefill, 0)

  cps = []
  for r in range(P1_GRPS):
    cps.append(pltpu.async_copy(jvs_v.at[r], w_sh.at[idxs_v.at[r]], sem_a))
  for cp in cps:
    cp.wait()
  plsc.subcore_barrier()

  cps = []
  for r in range(P1_GRPS):
    cps.append(pltpu.async_copy(w_sh.at[idxs_v.at[r]], wjs_v.at[r], sem_b))
  for cp in cps:
    cp.wait()

  def compact(i, cnt):
    r = i // 8
    col = (i % 8) * 16
    kv = idxs_v[r, pl.ds(col, 16)]
    jv = jvs_v[r, pl.ds(col, 16)]
    wv = wjs_v[r, pl.ds(col, 16)]
    m = wv < jv
    mi = m.astype(jnp.int32)
    excl = plsc.cumsum(mi) - mi
    dest = excl + cnt
    plsc.store_scatter(fk_v, [dest], kv, mask=m)
    plsc.store_scatter(fj_v, [dest], jv, mask=m)
    return cnt + jnp.sum(mi)
  cnt = lax.fori_loop(0, P1_CHUNK // 16, compact, 0)
  ngrp = (cnt + GR - 1) // GR

  for t in range(NS):
    @pl.when(s == t)
    def _():
      def fix(g, carry):
        for k in range(GR // 16):
          stg_v[0, pl.ds(k * 16, 16)] = fk_v[pl.ds(g * GR + k * 16, 16)]
        pltpu.sync_copy(fj_v.at[pl.ds(g * GR, GR)], w_sh.at[stg_v.at[0]])
        return carry
      lax.fori_loop(0, ngrp, fix, 0)
    plsc.subcore_barrier()

  w = s * NC + c
  pltpu.sync_copy(idx_hbm.at[pl.ds(w * P2_GRPS, P2_GRPS)], oidx_v)
  cps = []
  for r in range(P2_GRPS):
    cps.append(pltpu.async_copy(w_sh.at[oidx_v.at[r]], wj_v.at[r], sem_a))
  for cp in cps:
    cp.wait()
  pltpu.sync_copy(wj_v, wj_hbm.at[pl.ds(w * P2_GRPS, P2_GRPS)])


def _permute_body(wj_hbm, val2_hbm, out_hbm,
                  wj_v, pidx_v, p0_v, p1_v, p2_v, p3_v, ob0_v, ob1_v,
                  gsems, wsems):
  c = lax.axis_index("c")
  s = lax.axis_index("s")
  w = s * NC + c

  pltpu.sync_copy(wj_hbm.at[pl.ds(w * P2_GRPS, P2_GRPS)], wj_v)

  def mkpidx(i, carry):
    r = i // 8
    col = (i % 8) * 16
    pidx_v[r, pl.ds(col, 16)] = jax.lax.shift_right_logical(
        wj_v[r, pl.ds(col, 16)], 1)
    return carry
  lax.fori_loop(0, P2_CHUNK // 16, mkpidx, 0)

  pbufs = [p0_v, p1_v, p2_v, p3_v]
  obufs = [ob0_v, ob1_v]

  def select_group(g, pbuf, obuf):
    def blk(b, carry):
      offv = (wj_v[g, pl.ds(b * 16, 16)] & 1) * D
      for l in range(16):
        i = b * 16 + l
        off = offv[l]
        for k in range(D // 16):
          obuf[i, pl.ds(k * 16, 16)] = pbuf[i, pl.ds(off + k * 16, 16)]
      return carry
    lax.fori_loop(0, GR // 16, blk, 0)

  def finish_group(gp):
    gcps[gp].wait()
    if gp >= 2:
      wcps[gp - 2].wait()
    select_group(gp, pbufs[gp % NBUF], obufs[gp % 2])
    wcps[gp] = pltpu.async_copy(
        obufs[gp % 2],
        out_hbm.at[pl.ds(w * P2_CHUNK + gp * GR, GR)],
        wsems.at[gp % 2])

  gcps = [None] * P2_GRPS
  wcps = [None] * P2_GRPS
  for g in range(P2_GRPS):
    gcps[g] = pltpu.async_copy(
        val2_hbm.at[pidx_v.at[g]], pbufs[g % NBUF], gsems.at[g % NBUF])
    if g >= 2:
      finish_group(g - 2)
  finish_group(P2_GRPS - 2)
  finish_group(P2_GRPS - 1)
  wcps[P2_GRPS - 2].wait()
  wcps[P2_GRPS - 1].wait()


@jax.jit
def _run(idx2d, jv2d, val2):
  mesh = plsc.VectorSubcoreMesh(core_axis_name="c", subcore_axis_name="s")
  params = pltpu.CompilerParams(needs_layout_passes=False)
  winners = functools.partial(
      pl.kernel,
      mesh=mesh,
      compiler_params=params,
      out_type=jax.ShapeDtypeStruct((NGRP, GR), jnp.int32),
      scratch_types=[
          pltpu.VMEM((P1_GRPS, GR), jnp.int32),
          pltpu.VMEM((P1_GRPS, GR), jnp.int32),
          pltpu.VMEM((P1_GRPS, GR), jnp.int32),
          pltpu.VMEM((FLEN,), jnp.int32),
          pltpu.VMEM((FLEN,), jnp.int32),
          pltpu.VMEM((1, GR), jnp.int32),
          pltpu.VMEM((P2_GRPS, GR), jnp.int32),
          pltpu.VMEM((P2_GRPS, GR), jnp.int32),
          pltpu.VMEM_SHARED((M + WPAD,), jnp.int32),
          pltpu.SemaphoreType.DMA,
          pltpu.SemaphoreType.DMA,
      ],
  )(_winners_body)
  permute = functools.partial(
      pl.kernel,
      mesh=mesh,
      compiler_params=params,
      out_type=jax.ShapeDtypeStruct((B, D), jnp.float32),
      scratch_types=[
          pltpu.VMEM((P2_GRPS, GR), jnp.int32),
          pltpu.VMEM((P2_GRPS, GR), jnp.int32),
          pltpu.VMEM((GR, 2 * D), jnp.float32),
          pltpu.VMEM((GR, 2 * D), jnp.float32),
          pltpu.VMEM((GR, 2 * D), jnp.float32),
          pltpu.VMEM((GR, 2 * D), jnp.float32),
          pltpu.VMEM((GR, D), jnp.float32),
          pltpu.VMEM((GR, D), jnp.float32),
          pltpu.SemaphoreType.DMA((NBUF,)),
          pltpu.SemaphoreType.DMA((2,)),
      ],
  )(_permute_body)
  wj2d = winners(idx2d, jv2d)
  return permute(wj2d, val2)


def kernel(mem, idx, val):
  del mem
  idx2d = idx.reshape(NGRP, GR)
  jv2d = jnp.arange(B, dtype=jnp.int32).reshape(NGRP, GR)
  val2 = val.reshape(B // 2, 2 * D)
  return _run(idx2d, jv2d, val2)
